# baseline (speedup 1.0000x reference)
import jax
import jax.numpy as jnp
from jax.experimental import pallas as pl
from jax.experimental.pallas import tpu as pltpu

POOL = 2
ROWS_PER_BLOCK = 8


def _unpool_block(x_ref, o_ref):
    o_ref[...] = jnp.repeat(x_ref[...], POOL, axis=1)


def kernel(x):
    b, c, n = x.shape
    rows = b * c
    x2 = x.reshape(rows, n)
    grid = rows // ROWS_PER_BLOCK
    out = pl.pallas_call(
        _unpool_block,
        grid=(grid,),
        in_specs=[pl.BlockSpec((ROWS_PER_BLOCK, n), lambda i: (i, 0))],
        out_specs=pl.BlockSpec((ROWS_PER_BLOCK, POOL * n), lambda i: (i, 0)),
        out_shape=jax.ShapeDtypeStruct((rows, POOL * n), x.dtype),
    )(x2)
    return out.reshape(b, c, POOL * n)


# speedup vs baseline: 12.4812x; 12.4812x over previous
import functools

import jax
import jax.numpy as jnp
from jax import lax
from jax.experimental import pallas as pl
from jax.experimental.pallas import tpu as pltpu
from jax.experimental.pallas import tpu_sc as plsc

POOL = 2
NUM_CORES = 2
NUM_SUBCORES = 16
LANES = 16
NUM_WORKERS = NUM_CORES * NUM_SUBCORES

CHUNK = 16384
UNROLL = 8


def _sc_body(x_hbm, o_hbm, in0, in1, out0, out1, si0, si1, so0, so1):
    n_total = x_hbm.shape[0]
    span = n_total // NUM_WORKERS
    nch = span // CHUNK

    wid = lax.axis_index("s") * NUM_CORES + lax.axis_index("c")
    base = wid * span

    ins = (in0, in1)
    outs = (out0, out1)
    sis = (si0, si1)
    sos = (so0, so1)

    iota = lax.iota(jnp.int32, LANES)
    idx_even0 = POOL * iota
    idx_odd0 = POOL * iota + 1

    def in_copy(t, slot):
        return pltpu.async_copy(
            x_hbm.at[pl.ds(base + t * CHUNK, CHUNK)], ins[slot], sis[slot]
        )

    def out_copy(t, slot):
        return pltpu.async_copy(
            outs[slot],
            o_hbm.at[pl.ds(POOL * (base + t * CHUNK), POOL * CHUNK)],
            sos[slot],
        )

    in_handles = [in_copy(0, 0), None]
    out_handles = [None, None]

    for t in range(nch):
        cur = t % 2
        nxt = (t + 1) % 2
        if t + 1 < nch:
            in_handles[nxt] = in_copy(t + 1, nxt)
        in_handles[cur].wait()
        if out_handles[cur] is not None:
            out_handles[cur].wait()

        inb = ins[cur]
        outb = outs[cur]

        def body(i, carry):
            idxe, idxo = carry
            b = i * (UNROLL * LANES)
            for u in range(UNROLL):
                v = inb[pl.ds(b + u * LANES, LANES)]
                plsc.store_scatter(outb, [idxe], v)
                plsc.store_scatter(outb, [idxo], v)
                idxe = idxe + POOL * LANES
                idxo = idxo + POOL * LANES
            return (idxe, idxo)

        lax.fori_loop(
            0, CHUNK // (UNROLL * LANES), body, (idx_even0, idx_odd0)
        )

        out_handles[cur] = out_copy(t, cur)

    for h in out_handles:
        if h is not None:
            h.wait()


def kernel(x):
    b, c, n = x.shape
    total = b * c * n
    flat = x.reshape(total)

    mesh = plsc.VectorSubcoreMesh(
        core_axis_name="c",
        subcore_axis_name="s",
        num_cores=NUM_CORES,
        num_subcores=NUM_SUBCORES,
    )
    run = pl.kernel(
        _sc_body,
        out_type=jax.ShapeDtypeStruct((POOL * total,), jnp.float32),
        mesh=mesh,
        compiler_params=pltpu.CompilerParams(needs_layout_passes=False),
        scratch_types=[
            pltpu.VMEM((CHUNK,), jnp.float32),
            pltpu.VMEM((CHUNK,), jnp.float32),
            pltpu.VMEM((POOL * CHUNK,), jnp.float32),
            pltpu.VMEM((POOL * CHUNK,), jnp.float32),
            pltpu.SemaphoreType.DMA,
            pltpu.SemaphoreType.DMA,
            pltpu.SemaphoreType.DMA,
            pltpu.SemaphoreType.DMA,
        ],
    )
    out = run(flat)
    return out.reshape(b, c, POOL * n)


# speedup vs baseline: 15.4716x; 1.2396x over previous
import functools

import jax
import jax.numpy as jnp
from jax import lax
from jax.experimental import pallas as pl
from jax.experimental.pallas import tpu as pltpu
from jax.experimental.pallas import tpu_sc as plsc

POOL = 2
NUM_CORES = 2
NUM_SUBCORES = 16
LANES = 16
NUM_WORKERS = NUM_CORES * NUM_SUBCORES

CHUNK = 16384
UNROLL = 8


def _sc_body(x_hbm, o_hbm, in0, in1, out0, out1, si0, si1, so0, so1):
    n_total = x_hbm.shape[0]
    span = n_total // NUM_WORKERS
    nch = span // CHUNK

    wid = lax.axis_index("s") * NUM_CORES + lax.axis_index("c")
    base = wid * span

    ins = (in0, in1)
    outs = (out0, out1)
    sis = (si0, si1)
    sos = (so0, so1)

    iota = lax.iota(jnp.int32, LANES)
    idx_even0 = POOL * iota
    idx_odd0 = POOL * iota + 1

    def in_copy(t, slot):
        return pltpu.async_copy(
            x_hbm.at[pl.ds(base + t * CHUNK, CHUNK)], ins[slot], sis[slot]
        )

    def out_copy(t, slot):
        return pltpu.async_copy(
            outs[slot],
            o_hbm.at[pl.ds(POOL * (base + t * CHUNK), POOL * CHUNK)],
            sos[slot],
        )

    in_handles = [in_copy(0, 0), None]
    out_handles = [None, None]

    for t in range(nch):
        cur = t % 2
        nxt = (t + 1) % 2
        if t + 1 < nch:
            in_handles[nxt] = in_copy(t + 1, nxt)
        in_handles[cur].wait()
        if out_handles[cur] is not None:
            out_handles[cur].wait()

        inb = ins[cur]
        outb = outs[cur]

        def body(i, carry):
            idxe, idxo = carry
            b = i * (UNROLL * LANES)
            vs = [inb[pl.ds(b + u * LANES, LANES)] for u in range(UNROLL)]
            for u in range(UNROLL):
                plsc.store_scatter(outb, [idxe], vs[u])
                plsc.store_scatter(outb, [idxo], vs[u])
                idxe = idxe + POOL * LANES
                idxo = idxo + POOL * LANES
            return (idxe, idxo)

        lax.fori_loop(
            0, CHUNK // (UNROLL * LANES), body, (idx_even0, idx_odd0)
        )

        out_handles[cur] = out_copy(t, cur)

    for h in out_handles:
        if h is not None:
            h.wait()


def kernel(x):
    b, c, n = x.shape
    total = b * c * n
    flat = x.reshape(total)

    mesh = plsc.VectorSubcoreMesh(
        core_axis_name="c",
        subcore_axis_name="s",
        num_cores=NUM_CORES,
        num_subcores=NUM_SUBCORES,
    )
    run = pl.kernel(
        _sc_body,
        out_type=jax.ShapeDtypeStruct((POOL * total,), jnp.float32),
        mesh=mesh,
        compiler_params=pltpu.CompilerParams(needs_layout_passes=False),
        scratch_types=[
            pltpu.VMEM((CHUNK,), jnp.float32),
            pltpu.VMEM((CHUNK,), jnp.float32),
            pltpu.VMEM((POOL * CHUNK,), jnp.float32),
            pltpu.VMEM((POOL * CHUNK,), jnp.float32),
            pltpu.SemaphoreType.DMA,
            pltpu.SemaphoreType.DMA,
            pltpu.SemaphoreType.DMA,
            pltpu.SemaphoreType.DMA,
        ],
    )
    out = run(flat)
    return out.reshape(b, c, POOL * n)


# speedup vs baseline: 42.7367x; 2.7623x over previous
import jax
import jax.numpy as jnp
from jax import lax
from jax.experimental import pallas as pl
from jax.experimental.pallas import tpu as pltpu
from jax.experimental.pallas import tpu_sc as plsc

POOL = 2
NUM_CORES = 2
NUM_SUBCORES = 16
LANES = 16
NUM_WORKERS = NUM_CORES * NUM_SUBCORES

ROWS = 4096
COLS = 4096
CHUNK_ROWS = 8
CHUNK_COLS = 2048
UNROLL = 8


def _sc_body(x_hbm, o_hbm, in0, in1, out0, out1, si0, si1, so0, so1):
    rows_per_worker = ROWS // NUM_WORKERS
    col_halves = COLS // CHUNK_COLS
    nch = (rows_per_worker // CHUNK_ROWS) * col_halves

    wid = lax.axis_index("s") * NUM_CORES + lax.axis_index("c")
    row0 = wid * rows_per_worker

    ins = (in0, in1)
    outs = (out0, out1)
    sis = (si0, si1)
    sos = (so0, so1)

    iota = lax.iota(jnp.int32, LANES)
    idx_even0 = POOL * iota
    idx_odd0 = POOL * iota + 1

    def chunk_slices(t):
        r = row0 + (t // col_halves) * CHUNK_ROWS
        c = (t % col_halves) * CHUNK_COLS
        return (pl.ds(r, CHUNK_ROWS), pl.ds(c, CHUNK_COLS)), (
            pl.ds(r, CHUNK_ROWS),
            pl.ds(POOL * c, POOL * CHUNK_COLS),
        )

    def in_copy(t, slot):
        (rs, cs), _ = chunk_slices(t)
        return pltpu.async_copy(x_hbm.at[rs, cs], ins[slot], sis[slot])

    def out_copy(t, slot):
        _, (rs, cs) = chunk_slices(t)
        return pltpu.async_copy(outs[slot], o_hbm.at[rs, cs], sos[slot])

    in_handles = [in_copy(0, 0), None]
    out_handles = [None, None]

    for t in range(nch):
        cur = t % 2
        nxt = (t + 1) % 2
        if t + 1 < nch:
            in_handles[nxt] = in_copy(t + 1, nxt)
        in_handles[cur].wait()
        if out_handles[cur] is not None:
            out_handles[cur].wait()

        inb = ins[cur]
        outb = outs[cur]

        def srow(s, _):
            svec = jnp.full((LANES,), s, jnp.int32)

            def body(i, carry):
                idxe, idxo = carry
                b = i * (UNROLL * LANES)
                vs = [
                    inb[s, pl.ds(b + u * LANES, LANES)] for u in range(UNROLL)
                ]
                for u in range(UNROLL):
                    plsc.store_scatter(outb, [svec, idxe], vs[u])
                    plsc.store_scatter(outb, [svec, idxo], vs[u])
                    idxe = idxe + POOL * LANES
                    idxo = idxo + POOL * LANES
                return (idxe, idxo)

            lax.fori_loop(
                0,
                CHUNK_COLS // (UNROLL * LANES),
                body,
                (idx_even0, idx_odd0),
            )
            return 0

        lax.fori_loop(0, CHUNK_ROWS, srow, 0)

        out_handles[cur] = out_copy(t, cur)

    for h in out_handles:
        if h is not None:
            h.wait()


def kernel(x):
    b, c, n = x.shape
    x2 = x.reshape(ROWS, COLS)

    mesh = plsc.VectorSubcoreMesh(
        core_axis_name="c",
        subcore_axis_name="s",
        num_cores=NUM_CORES,
        num_subcores=NUM_SUBCORES,
    )
    run = pl.kernel(
        _sc_body,
        out_type=jax.ShapeDtypeStruct((ROWS, POOL * COLS), jnp.float32),
        mesh=mesh,
        compiler_params=pltpu.CompilerParams(needs_layout_passes=False),
        scratch_types=[
            pltpu.VMEM((CHUNK_ROWS, CHUNK_COLS), jnp.float32),
            pltpu.VMEM((CHUNK_ROWS, CHUNK_COLS), jnp.float32),
            pltpu.VMEM((CHUNK_ROWS, POOL * CHUNK_COLS), jnp.float32),
            pltpu.VMEM((CHUNK_ROWS, POOL * CHUNK_COLS), jnp.float32),
            pltpu.SemaphoreType.DMA,
            pltpu.SemaphoreType.DMA,
            pltpu.SemaphoreType.DMA,
            pltpu.SemaphoreType.DMA,
        ],
    )
    out = run(x2)
    return out.reshape(b, c, POOL * n)


# speedup vs baseline: 44.5795x; 1.0431x over previous
import jax
import jax.numpy as jnp
from jax import lax
from jax.experimental import pallas as pl
from jax.experimental.pallas import tpu as pltpu
from jax.experimental.pallas import tpu_sc as plsc

POOL = 2
NUM_CORES = 2
NUM_SUBCORES = 16
LANES = 16
NUM_WORKERS = NUM_CORES * NUM_SUBCORES

ROWS = 4096
COLS = 4096
CHUNK_ROWS = 8
CHUNK_COLS = 2048
CHUNK = CHUNK_ROWS * CHUNK_COLS
TILE = 128
NSEG = CHUNK // TILE


def _sc_body(x_hbm, o_hbm, in0, in1, out0, out1, si0, si1, so0, so1):
    rows_per_worker = ROWS // NUM_WORKERS
    col_halves = COLS // CHUNK_COLS
    nch = (rows_per_worker // CHUNK_ROWS) * col_halves

    wid = lax.axis_index("s") * NUM_CORES + lax.axis_index("c")
    row0 = wid * rows_per_worker

    ins = (in0, in1)
    outs = (out0, out1)
    sis = (si0, si1)
    sos = (so0, so1)

    iota = lax.iota(jnp.int32, LANES)
    lanes_e = [POOL * LANES * (u % 4) + POOL * iota for u in range(8)]
    lanes_o = [POOL * LANES * (u % 4) + POOL * iota + 1 for u in range(8)]

    def chunk_slices(t):
        r = row0 + (t // col_halves) * CHUNK_ROWS
        c = (t % col_halves) * CHUNK_COLS
        return (pl.ds(r, CHUNK_ROWS), pl.ds(c, CHUNK_COLS)), (
            pl.ds(r, CHUNK_ROWS),
            pl.ds(POOL * c, POOL * CHUNK_COLS),
        )

    def in_copy(t, slot):
        (rs, cs), _ = chunk_slices(t)
        return pltpu.async_copy(x_hbm.at[rs, cs], ins[slot], sis[slot])

    def out_copy(t, slot):
        _, (rs, cs) = chunk_slices(t)
        return pltpu.async_copy(outs[slot], o_hbm.at[rs, cs], sos[slot])

    in_handles = [in_copy(0, 0), None]
    out_handles = [None, None]

    for t in range(nch):
        cur = t % 2
        nxt = (t + 1) % 2
        if t + 1 < nch:
            in_handles[nxt] = in_copy(t + 1, nxt)
        in_handles[cur].wait()
        if out_handles[cur] is not None:
            out_handles[cur].wait()

        inb = ins[cur]
        outb = outs[cur]

        def body(k, _):
            ct = k >> 3
            s = k & 7
            cin = pl.multiple_of(ct * TILE, TILE)
            clo = pl.multiple_of(POOL * ct * TILE, TILE)
            row_in = inb.at[s, pl.ds(cin, TILE)]
            row_lo = outb.at[s, pl.ds(clo, TILE)]
            row_hi = outb.at[s, pl.ds(clo + TILE, TILE)]
            vs = [row_in[pl.ds(u * LANES, LANES)] for u in range(8)]
            for u in range(8):
                w = row_lo if u < 4 else row_hi
                plsc.store_scatter(w, [lanes_e[u]], vs[u])
                plsc.store_scatter(w, [lanes_o[u]], vs[u])
            return 0

        lax.fori_loop(0, NSEG, body, 0)

        out_handles[cur] = out_copy(t, cur)

    for h in out_handles:
        if h is not None:
            h.wait()


def kernel(x):
    b, c, n = x.shape
    x2 = x.reshape(ROWS, COLS)

    mesh = plsc.VectorSubcoreMesh(
        core_axis_name="c",
        subcore_axis_name="s",
        num_cores=NUM_CORES,
        num_subcores=NUM_SUBCORES,
    )
    run = pl.kernel(
        _sc_body,
        out_type=jax.ShapeDtypeStruct((ROWS, POOL * COLS), jnp.float32),
        mesh=mesh,
        compiler_params=pltpu.CompilerParams(needs_layout_passes=False),
        scratch_types=[
            pltpu.VMEM((CHUNK_ROWS, CHUNK_COLS), jnp.float32),
            pltpu.VMEM((CHUNK_ROWS, CHUNK_COLS), jnp.float32),
            pltpu.VMEM((CHUNK_ROWS, POOL * CHUNK_COLS), jnp.float32),
            pltpu.VMEM((CHUNK_ROWS, POOL * CHUNK_COLS), jnp.float32),
            pltpu.SemaphoreType.DMA,
            pltpu.SemaphoreType.DMA,
            pltpu.SemaphoreType.DMA,
            pltpu.SemaphoreType.DMA,
        ],
    )
    out = run(x2)
    return out.reshape(b, c, POOL * n)
